# Initial kernel scaffold; baseline (speedup 1.0000x reference)
import jax
import jax.numpy as jnp
from jax.experimental import pallas as pl


def kernel(x_2d, edge_index_2d, edge_attr_2d, batch_2d, z_3d, pos_3d, batch_3d, edge_index_3d, params):
    raise NotImplementedError("write your pallas kernel here")


# speedup vs baseline: 1.3907x; 1.3907x over previous
import jax, jax.numpy as jnp
from jax.experimental import pallas as pl

N, E, G, D, H, NG, L, NI, CUT, NT = 10000, 320000, 256, 128, 128, 50, 3, 3, 10.0, 12

def _ssp(x):
    return jax.nn.softplus(x) - jnp.log(2.0)

def forward2(x_2d, ei2, ea2, b2, z, pos, b3, ei3, p):
    Nn = x_2d.shape[0]
    s2, d2 = ei2[0], ei2[1]
    atomcomb = p['atom_emb1'][:, None, :] + p['atom_emb2'][None, :, :]
    atomcomb = atomcomb.reshape(120 * 3, D)
    h = atomcomb[x_2d[:, 0] * 3 + x_2d[:, 1]]
    t = ea2[:, 0] * 3 + ea2[:, 1]
    eye32 = jnp.eye(32, dtype=jnp.float32)[:18]
    cnt = jax.ops.segment_sum(eye32[t], d2, num_segments=Nn)
    vn = jnp.zeros((G, D), jnp.float32)
    for l in range(L):
        comb = p['gin%d_bond1' % l][:, None, :] + p['gin%d_bond2' % l][None, :, :]
        comb32 = jnp.zeros((32, D), jnp.float32).at[:18].set(comb.reshape(18, D))
        hin = h + vn[b2]
        seg = jax.ops.segment_sum(hin[s2], d2, num_segments=Nn)
        agg = seg + hin + cnt @ comb32
        hmid = jax.nn.relu(agg @ p['gin%d_W1' % l] + p['gin%d_b1' % l])
        h = hmid @ p['gin%d_W2' % l] + p['gin%d_b2' % l]
        if l < L - 1:
            h = jax.nn.relu(h)
            vnseg = jax.ops.segment_sum(h, b2, num_segments=G)
            vn_in = vnseg + vn
            vmid = jax.nn.relu(vn_in @ p['vn%d_W1' % l] + p['vn%d_b1' % l])
            vn = jax.nn.relu(vmid @ p['vn%d_W2' % l] + p['vn%d_b2' % l])
    cntb = jax.ops.segment_sum(jnp.ones((Nn,), jnp.float32), b2, num_segments=G)
    cntb = jnp.clip(cntb, 1.0)[:, None]
    h2d = jax.ops.segment_sum(h, b2, num_segments=G) / cntb
    hs = p['z_emb'][z]
    s3, d3 = ei3[0], ei3[1]
    dvec = pos[s3] - pos[d3]
    dist2 = jnp.sum(dvec * dvec, axis=1) + 1e-12
    dist = jnp.sqrt(dist2)
    offsets = jnp.linspace(0.0, CUT, NG)
    coeff = -0.5 / (offsets[1] - offsets[0]) ** 2
    ea3 = jnp.exp(coeff * (dist[:, None] - offsets[None, :]) ** 2)
    Cw = 0.5 * (jnp.cos(dist * jnp.pi / CUT) + 1.0) * (dist < CUT).astype(jnp.float32)
    for i in range(NI):
        Wf = _ssp(ea3 @ p['sch%d_mW1' % i] + p['sch%d_mb1' % i]) @ p['sch%d_mW2' % i] + p['sch%d_mb2' % i]
        Wf = Wf * Cw[:, None]
        x1 = hs @ p['sch%d_lin1' % i]
        agg = jax.ops.segment_sum(x1[s3] * Wf, d3, num_segments=Nn)
        x2 = _ssp(agg @ p['sch%d_lin2W' % i] + p['sch%d_lin2b' % i])
        x2 = x2 @ p['sch%d_linW' % i] + p['sch%d_linb' % i]
        hs = hs + x2
    h3d = jax.ops.segment_sum(hs, b3, num_segments=G) / cntb
    h2p = h2d @ p['proj2d_W'] + p['proj2d_b']
    h3p = h3d @ p['proj3d_W'] + p['proj3d_b']
    cat = jnp.concatenate([h2p, h3p], axis=1)
    a = jax.nn.sigmoid(jax.nn.relu(cat @ p['gate_W1'] + p['gate_b1']) @ p['gate_W2'] + p['gate_b2'])
    hf = a * h2p + (1.0 - a) * h3p
    out = jax.nn.relu(hf @ p['cls_W1'] + p['cls_b1']) @ p['cls_W2'] + p['cls_b2']
    return out


def kernel(x_2d, edge_index_2d, edge_attr_2d, batch_2d, z_3d, pos_3d, batch_3d, edge_index_3d, params):
    return forward2(x_2d, edge_index_2d, edge_attr_2d, batch_2d, z_3d, pos_3d, batch_3d, edge_index_3d, params)


# speedup vs baseline: 2.4493x; 1.7612x over previous
import functools
import math

import jax
import jax.numpy as jnp
from jax import lax
from jax.experimental import pallas as pl
from jax.experimental.pallas import tpu as pltpu
from jax.experimental.pallas import tpu_sc as plsc

N = 10000
E = 320000
G = 256
D = 128
H = 128
NG = 50
L = 3
NI = 3
CUT = 10.0
NT = 12

NC = 2
NS = 16
NW = NC * NS
NP = 10240
RPW = NP // NW
EPW = E // NW
GP = 512

_MESH = plsc.VectorSubcoreMesh(core_axis_name="c", subcore_axis_name="s")


def _wid():
    return lax.axis_index("s") * NC + lax.axis_index("c")


def _zero_rows(ref, nrows, d):
    zeros = jnp.zeros((16,), jnp.float32)
    nk = d // 16

    def body(r, _):
        for k in range(nk):
            ref[r, pl.ds(k * 16, 16)] = zeros
        return 0

    lax.fori_loop(0, nrows, body, 0)


def _rows_binop(dst, src, nrows, d, mul):
    nk = d // 16

    def body(r, _):
        for k in range(nk):
            sl = pl.ds(k * 16, 16)
            a = dst[r, sl]
            b = src[r, sl]
            dst[r, sl] = a * b if mul else a + b
        return 0

    lax.fori_loop(0, nrows, body, 0)


def _zero_acc_and_barrier(acc, stage, noutp, d):
    sid = lax.axis_index("s")
    per = noutp // NS
    srows = stage.shape[0]
    _zero_rows(stage, min(per, srows), d)
    base = sid * per
    done = 0
    while done < per:
        n = min(srows, per - done)
        pltpu.sync_copy(stage.at[pl.ds(0, n)], acc.at[pl.ds(base + done, n)])
        done += n
    plsc.subcore_barrier()


def _acc_readout(acc, stage, out, cid, noutp, d):
    sid = lax.axis_index("s")
    per = noutp // NS
    srows = stage.shape[0]
    base = sid * per
    done = 0
    while done < per:
        n = min(srows, per - done)
        pltpu.sync_copy(acc.at[pl.ds(base + done, n)], stage.at[pl.ds(0, n)])
        pltpu.sync_copy(stage.at[pl.ds(0, n)], out.at[cid, pl.ds(base + done, n)])
        done += n


@functools.partial(jax.jit, static_argnames=("with_base",))
def sc_gather(table, idx, base=None, *, with_base=False):
    d = table.shape[1]
    scratch = [
        pltpu.VMEM((RPW,), jnp.int32),
        pltpu.VMEM((RPW, d), jnp.float32),
        pltpu.SemaphoreType.DMA,
    ]
    if with_base:
        scratch.insert(2, pltpu.VMEM((RPW, d), jnp.float32))

    def body(*refs):
        if with_base:
            table_r, idx_r, base_r, out_r, idxv, rows, bbuf, sem = refs
        else:
            table_r, idx_r, out_r, idxv, rows, sem = refs
        r0 = _wid() * RPW
        pltpu.sync_copy(idx_r.at[pl.ds(r0, RPW)], idxv)
        pltpu.async_copy(table_r.at[idxv], rows, sem).wait()
        if with_base:
            pltpu.sync_copy(base_r.at[pl.ds(r0, RPW)], bbuf)
            _rows_binop(rows, bbuf, RPW, d, mul=False)
        pltpu.sync_copy(rows, out_r.at[pl.ds(r0, RPW)])

    fn = pl.kernel(
        body,
        out_type=jax.ShapeDtypeStruct((NP, d), jnp.float32),
        mesh=_MESH,
        scratch_types=scratch,
    )
    return fn(table, idx, base) if with_base else fn(table, idx)


@functools.partial(jax.jit, static_argnames=("with_wf",))
def sc_segsum_gather(table, src, dst, wf=None, *, with_wf=False):
    d = table.shape[1]
    EC = 80 if with_wf else 200
    NCH = EPW // EC
    scratch = [
        pltpu.VMEM((EC,), jnp.int32),
        pltpu.VMEM((EC,), jnp.int32),
        pltpu.VMEM((EC, d), jnp.float32),
        pltpu.VMEM_SHARED((NP, d), jnp.float32),
        pltpu.SemaphoreType.DMA,
    ]
    if with_wf:
        scratch.insert(3, pltpu.VMEM((EC, d), jnp.float32))

    def body(*refs):
        if with_wf:
            table_r, src_r, dst_r, wf_r, out_r, sidx, didx, rows, wfb, acc, sem = refs
        else:
            table_r, src_r, dst_r, out_r, sidx, didx, rows, acc, sem = refs
        cid = lax.axis_index("c")
        wid = _wid()
        _zero_acc_and_barrier(acc, rows, NP, d)
        e0 = wid * EPW

        def chunk(j, _):
            base = e0 + j * EC
            pltpu.sync_copy(src_r.at[pl.ds(base, EC)], sidx)
            pltpu.async_copy(table_r.at[sidx], rows, sem).wait()
            if with_wf:
                pltpu.sync_copy(wf_r.at[pl.ds(base, EC)], wfb)
                _rows_binop(rows, wfb, EC, d, mul=True)
            pltpu.sync_copy(dst_r.at[pl.ds(base, EC)], didx)
            pltpu.sync_copy(rows, acc.at[didx], add=True)
            return 0

        lax.fori_loop(0, NCH, chunk, 0)
        plsc.subcore_barrier()
        _acc_readout(acc, rows, out_r, cid, NP, d)

    fn = pl.kernel(
        body,
        out_type=jax.ShapeDtypeStruct((NC, NP, d), jnp.float32),
        mesh=_MESH,
        scratch_types=scratch,
    )
    return fn(table, src, dst, wf) if with_wf else fn(table, src, dst)


@jax.jit
def sc_scatter_rows(values, dst):
    d = values.shape[1]

    def body(val_r, dst_r, out_r, didx, rows, acc, sem):
        cid = lax.axis_index("c")
        wid = _wid()
        _zero_acc_and_barrier(acc, rows, GP, d)
        r0 = wid * RPW
        pltpu.sync_copy(val_r.at[pl.ds(r0, RPW)], rows)
        pltpu.sync_copy(dst_r.at[pl.ds(r0, RPW)], didx)
        pltpu.sync_copy(rows, acc.at[didx], add=True)
        plsc.subcore_barrier()
        _acc_readout(acc, rows, out_r, cid, GP, d)

    fn = pl.kernel(
        body,
        out_type=jax.ShapeDtypeStruct((NC, GP, d), jnp.float32),
        mesh=_MESH,
        scratch_types=[
            pltpu.VMEM((RPW,), jnp.int32),
            pltpu.VMEM((RPW, d), jnp.float32),
            pltpu.VMEM_SHARED((GP, d), jnp.float32),
            pltpu.SemaphoreType.DMA,
        ],
    )
    return fn(values, dst)


@jax.jit
def sc_dist2(px, py, pz, src, dst):
    def body(px_r, py_r, pz_r, src_r, dst_r, out_r,
             sidx, didx, ax, ay, az, bx, by, bz, outb, sem):
        wid = _wid()
        e0 = wid * EPW
        pltpu.sync_copy(src_r.at[pl.ds(e0, EPW)], sidx)
        pltpu.sync_copy(dst_r.at[pl.ds(e0, EPW)], didx)
        for tab, idx, buf in ((px_r, sidx, ax), (py_r, sidx, ay), (pz_r, sidx, az),
                              (px_r, didx, bx), (py_r, didx, by), (pz_r, didx, bz)):
            pltpu.async_copy(tab.at[idx], buf, sem).wait()

        def step(i, _):
            sl = pl.ds(i * 16, 16)
            dx = ax[sl] - bx[sl]
            dy = ay[sl] - by[sl]
            dz = az[sl] - bz[sl]
            outb[sl] = dx * dx + dy * dy + dz * dz + 1e-12
            return 0

        lax.fori_loop(0, EPW // 16, step, 0)
        pltpu.sync_copy(outb, out_r.at[pl.ds(e0, EPW)])

    fn = pl.kernel(
        body,
        out_type=jax.ShapeDtypeStruct((E,), jnp.float32),
        mesh=_MESH,
        scratch_types=[
            pltpu.VMEM((EPW,), jnp.int32),
            pltpu.VMEM((EPW,), jnp.int32),
            pltpu.VMEM((EPW,), jnp.float32),
            pltpu.VMEM((EPW,), jnp.float32),
            pltpu.VMEM((EPW,), jnp.float32),
            pltpu.VMEM((EPW,), jnp.float32),
            pltpu.VMEM((EPW,), jnp.float32),
            pltpu.VMEM((EPW,), jnp.float32),
            pltpu.VMEM((EPW,), jnp.float32),
            pltpu.SemaphoreType.DMA,
        ],
    )
    return fn(px, py, pz, src, dst)


def _ssp(x):
    return jax.nn.softplus(x) - math.log(2.0)


def _dot(a, b):
    return jnp.dot(a, b, preferred_element_type=jnp.float32)


BN = 2048


@functools.partial(jax.jit, static_argnames=("relu_out",))
def tc_gin_layer(p0, p1, hin, c0, c1, comb, w1, b1, w2, b2, *, relu_out):
    def body(p0r, p1r, hr, c0r, c1r, cbr, w1r, b1r, w2r, b2r, outr):
        agg = p0r[...] + p1r[...] + hr[...] + _dot(c0r[...] + c1r[...], cbr[...])
        hmid = jax.nn.relu(_dot(agg, w1r[...]) + b1r[...])
        res = _dot(hmid, w2r[...]) + b2r[...]
        outr[...] = jax.nn.relu(res) if relu_out else res

    nb = NP // BN
    full = lambda i: (0, 0)
    row = lambda i: (i, 0)
    return pl.pallas_call(
        body,
        grid=(nb,),
        in_specs=[
            pl.BlockSpec((BN, D), row),
            pl.BlockSpec((BN, D), row),
            pl.BlockSpec((BN, D), row),
            pl.BlockSpec((BN, D), row),
            pl.BlockSpec((BN, D), row),
            pl.BlockSpec((D, D), full),
            pl.BlockSpec((D, 2 * D), full),
            pl.BlockSpec((1, 2 * D), full),
            pl.BlockSpec((2 * D, D), full),
            pl.BlockSpec((1, D), full),
        ],
        out_specs=pl.BlockSpec((BN, D), row),
        out_shape=jax.ShapeDtypeStruct((NP, D), jnp.float32),
    )(p0, p1, hin, c0, c1, comb, w1, b1, w2, b2)


@jax.jit
def tc_vn_mlp(s0, s1, vn, w1, b1, w2, b2):
    def body(s0r, s1r, vnr, w1r, b1r, w2r, b2r, outr):
        vin = s0r[...] + s1r[...] + vnr[...]
        vmid = jax.nn.relu(_dot(vin, w1r[...]) + b1r[...])
        outr[...] = jax.nn.relu(_dot(vmid, w2r[...]) + b2r[...])

    return pl.pallas_call(
        body,
        out_shape=jax.ShapeDtypeStruct((G, D), jnp.float32),
    )(s0, s1, vn, w1, b1, w2, b2)


BE = 4000


@jax.jit
def tc_filter(dist2, offs, mw1, mb1, mw2, mb2):
    coeff = -0.5 / (CUT / (NG - 1)) ** 2

    def body(dr, offr, w1r, b1r, w2r, b2r, outr):
        dist = jnp.sqrt(dr[...])
        ea = jnp.exp(coeff * (dist - offr[...]) ** 2)
        mid = _ssp(_dot(ea, w1r[...]) + b1r[...])
        wf = _dot(mid, w2r[...]) + b2r[...]
        cw = 0.5 * (jnp.cos(dist * (math.pi / CUT)) + 1.0)
        cw = cw * (dist < CUT).astype(jnp.float32)
        outr[...] = wf * cw

    nb = E // BE
    full = lambda i: (0, 0)
    row = lambda i: (i, 0)
    return pl.pallas_call(
        body,
        grid=(nb,),
        in_specs=[
            pl.BlockSpec((BE, 1), row),
            pl.BlockSpec((1, 64), full),
            pl.BlockSpec((64, H), full),
            pl.BlockSpec((1, H), full),
            pl.BlockSpec((H, H), full),
            pl.BlockSpec((1, H), full),
        ],
        out_specs=pl.BlockSpec((BE, H), row),
        out_shape=jax.ShapeDtypeStruct((E, H), jnp.float32),
    )(dist2, offs, mw1, mb1, mw2, mb2)


@jax.jit
def tc_matmul(x, w):
    def body(xr, wr, outr):
        outr[...] = _dot(xr[...], wr[...])

    nb = NP // BN
    return pl.pallas_call(
        body,
        grid=(nb,),
        in_specs=[
            pl.BlockSpec((BN, D), lambda i: (i, 0)),
            pl.BlockSpec((D, D), lambda i: (0, 0)),
        ],
        out_specs=pl.BlockSpec((BN, D), lambda i: (i, 0)),
        out_shape=jax.ShapeDtypeStruct((NP, D), jnp.float32),
    )(x, w)


@jax.jit
def tc_sch_update(a0, a1, hs, w1, b1, w2, b2):
    def body(a0r, a1r, hsr, w1r, b1r, w2r, b2r, outr):
        x2 = _ssp(_dot(a0r[...] + a1r[...], w1r[...]) + b1r[...])
        outr[...] = hsr[...] + _dot(x2, w2r[...]) + b2r[...]

    nb = NP // BN
    full = lambda i: (0, 0)
    row = lambda i: (i, 0)
    return pl.pallas_call(
        body,
        grid=(nb,),
        in_specs=[
            pl.BlockSpec((BN, H), row),
            pl.BlockSpec((BN, H), row),
            pl.BlockSpec((BN, H), row),
            pl.BlockSpec((H, H), full),
            pl.BlockSpec((1, H), full),
            pl.BlockSpec((H, H), full),
            pl.BlockSpec((1, H), full),
        ],
        out_specs=pl.BlockSpec((BN, H), row),
        out_shape=jax.ShapeDtypeStruct((NP, H), jnp.float32),
    )(a0, a1, hs, w1, b1, w2, b2)


@jax.jit
def tc_head(h2a, h2b, h3a, h3b, cb0, cb1,
            p2w, p2b, p3w, p3b, gw1, gb1, gw2, gb2, cw1, clb1, cw2, clb2):
    def body(h2ar, h2br, h3ar, h3br, c0r, c1r,
             p2wr, p2br, p3wr, p3br, gw1r, gb1r, gw2r, gb2r,
             cw1r, clb1r, cw2r, clb2r, outr):
        cnt = jnp.maximum((c0r[...] + c1r[...])[:, 0:1], 1.0)
        h2d = (h2ar[...] + h2br[...]) / cnt
        h3d = (h3ar[...] + h3br[...]) / cnt
        h2p = _dot(h2d, p2wr[...]) + p2br[...]
        h3p = _dot(h3d, p3wr[...]) + p3br[...]
        cat = jnp.concatenate([h2p, h3p], axis=1)
        gmid = jax.nn.relu(_dot(cat, gw1r[...]) + gb1r[...])
        a = jax.nn.sigmoid(_dot(gmid, gw2r[...]) + gb2r[...])
        hf = a * h2p + (1.0 - a) * h3p
        cm = jax.nn.relu(_dot(hf, cw1r[...]) + clb1r[...])
        outr[...] = _dot(cm, cw2r[...]) + clb2r[...]

    return pl.pallas_call(
        body,
        out_shape=jax.ShapeDtypeStruct((G, NT), jnp.float32),
    )(h2a, h2b, h3a, h3b, cb0, cb1,
      p2w, p2b, p3w, p3b, gw1, gb1, gw2, gb2, cw1, clb1, cw2, clb2)


def kernel(x_2d, edge_index_2d, edge_attr_2d, batch_2d, z_3d, pos_3d,
           batch_3d, edge_index_3d, params):
    p = params
    npad = NP - N

    atomcomb = (p['atom_emb1'][:, None, :] + p['atom_emb2'][None, :, :])
    atomcomb = jnp.pad(atomcomb.reshape(360, D), ((0, 8), (0, 0)))
    aidx = jnp.pad(x_2d[:, 0] * 3 + x_2d[:, 1], (0, npad), constant_values=360)
    aidx = aidx.astype(jnp.int32)
    s2 = edge_index_2d[0].astype(jnp.int32)
    d2 = edge_index_2d[1].astype(jnp.int32)
    s3 = edge_index_3d[0].astype(jnp.int32)
    d3 = edge_index_3d[1].astype(jnp.int32)
    t18 = (edge_attr_2d[:, 0] * 3 + edge_attr_2d[:, 1]).astype(jnp.int32)
    eye128 = jnp.eye(128, dtype=jnp.float32)
    b2g = jnp.pad(batch_2d, (0, npad)).astype(jnp.int32)
    b2s = jnp.pad(batch_2d, (0, npad), constant_values=G).astype(jnp.int32)
    ztab = jnp.pad(p['z_emb'], ((0, 12), (0, 0)))
    zidx = jnp.pad(z_3d, (0, npad), constant_values=100).astype(jnp.int32)
    pos_pad = jnp.pad(pos_3d, ((0, npad), (0, 0)))
    px = jnp.asarray(pos_pad[:, 0])
    py = jnp.asarray(pos_pad[:, 1])
    pz = jnp.asarray(pos_pad[:, 2])
    ones128 = jnp.ones((NP, D), jnp.float32)
    offs = jnp.linspace(0.0, CUT, NG).astype(jnp.float32)
    offs64 = jnp.pad(offs, (0, 64 - NG), constant_values=CUT).reshape(1, 64)

    def b2d(v):
        return v.reshape(1, -1)

    h = sc_gather(atomcomb, aidx)
    cnt = sc_segsum_gather(eye128, t18, d2)
    vn = jnp.zeros((G, D), jnp.float32)
    for l in range(L):
        comb = (p['gin%d_bond1' % l][:, None, :] + p['gin%d_bond2' % l][None, :, :])
        comb128 = jnp.pad(comb.reshape(18, D), ((0, 110), (0, 0)))
        if l == 0:
            hin = h
        else:
            hin = sc_gather(vn, b2g, h, with_base=True)
        seg = sc_segsum_gather(hin, s2, d2)
        h = tc_gin_layer(seg[0], seg[1], hin, cnt[0], cnt[1], comb128,
                         p['gin%d_W1' % l], b2d(p['gin%d_b1' % l]),
                         p['gin%d_W2' % l], b2d(p['gin%d_b2' % l]),
                         relu_out=(l < L - 1))
        if l < L - 1:
            vseg = sc_scatter_rows(h, b2s)
            vn = tc_vn_mlp(vseg[0, :G], vseg[1, :G], vn,
                           p['vn%d_W1' % l], b2d(p['vn%d_b1' % l]),
                           p['vn%d_W2' % l], b2d(p['vn%d_b2' % l]))
    poolh = sc_scatter_rows(h, b2s)
    cntb = sc_scatter_rows(ones128, b2s)

    hs = sc_gather(ztab, zidx)
    dist2 = sc_dist2(px, py, pz, s3, d3).reshape(E, 1)
    for i in range(NI):
        mw1 = jnp.pad(p['sch%d_mW1' % i], ((0, 64 - NG), (0, 0)))
        wf = tc_filter(dist2, offs64, mw1, b2d(p['sch%d_mb1' % i]),
                       p['sch%d_mW2' % i], b2d(p['sch%d_mb2' % i]))
        x1 = tc_matmul(hs, p['sch%d_lin1' % i])
        seg = sc_segsum_gather(x1, s3, d3, wf, with_wf=True)
        hs = tc_sch_update(seg[0], seg[1], hs,
                           p['sch%d_lin2W' % i], b2d(p['sch%d_lin2b' % i]),
                           p['sch%d_linW' % i], b2d(p['sch%d_linb' % i]))
    poolhs = sc_scatter_rows(hs, b2s)

    out = tc_head(poolh[0, :G], poolh[1, :G], poolhs[0, :G], poolhs[1, :G],
                  cntb[0, :G], cntb[1, :G],
                  p['proj2d_W'], b2d(p['proj2d_b']),
                  p['proj3d_W'], b2d(p['proj3d_b']),
                  p['gate_W1'], b2d(p['gate_b1']),
                  p['gate_W2'], b2d(p['gate_b2']),
                  p['cls_W1'], b2d(p['cls_b1']),
                  p['cls_W2'], b2d(p['cls_b2']))
    return out


# speedup vs baseline: 2.6932x; 1.0996x over previous
import functools
import math

import jax
import jax.numpy as jnp
from jax import lax
from jax.experimental import pallas as pl
from jax.experimental.pallas import tpu as pltpu
from jax.experimental.pallas import tpu_sc as plsc

N = 10000
E = 320000
G = 256
D = 128
H = 128
NG = 50
L = 3
NI = 3
CUT = 10.0
NT = 12

NC = 2
NS = 16
NW = NC * NS
NP = 10240
RPW = NP // NW
EPW = E // NW
GP = 512

_MESH = plsc.VectorSubcoreMesh(core_axis_name="c", subcore_axis_name="s")


def _wid():
    return lax.axis_index("s") * NC + lax.axis_index("c")


def _zero_rows(ref, nrows, d):
    zeros = jnp.zeros((16,), jnp.float32)
    nk = d // 16

    def body(r, _):
        for k in range(nk):
            ref[r, pl.ds(k * 16, 16)] = zeros
        return 0

    lax.fori_loop(0, nrows, body, 0)


def _rows_binop(dst, src, nrows, d, mul):
    nk = d // 16
    UR = 4 if nrows % 4 == 0 else 1

    def body(q, _):
        for u in range(UR):
            r = q * UR + u
            for k in range(nk):
                sl = pl.ds(k * 16, 16)
                a = dst[r, sl]
                b = src[r, sl]
                dst[r, sl] = a * b if mul else a + b
        return 0

    lax.fori_loop(0, nrows // UR, body, 0)


def _zero_acc_and_barrier(acc, stage, noutp, d):
    sid = lax.axis_index("s")
    per = noutp // NS
    srows = stage.shape[0]
    _zero_rows(stage, min(per, srows), d)
    base = sid * per
    done = 0
    while done < per:
        n = min(srows, per - done)
        pltpu.sync_copy(stage.at[pl.ds(0, n)], acc.at[pl.ds(base + done, n)])
        done += n
    plsc.subcore_barrier()


def _acc_readout(acc, stage, out, cid, noutp, d):
    del stage, d
    sid = lax.axis_index("s")
    per = noutp // NS
    base = sid * per
    pltpu.sync_copy(acc.at[pl.ds(base, per)], out.at[cid, pl.ds(base, per)])


@functools.partial(jax.jit, static_argnames=("with_base",))
def sc_gather(table, idx, base=None, *, with_base=False):
    d = table.shape[1]
    scratch = [
        pltpu.VMEM((RPW,), jnp.int32),
        pltpu.VMEM((RPW, d), jnp.float32),
        pltpu.SemaphoreType.DMA,
    ]
    if with_base:
        scratch.insert(2, pltpu.VMEM((RPW, d), jnp.float32))

    def body(*refs):
        if with_base:
            table_r, idx_r, base_r, out_r, idxv, rows, bbuf, sem = refs
        else:
            table_r, idx_r, out_r, idxv, rows, sem = refs
        r0 = _wid() * RPW
        pltpu.sync_copy(idx_r.at[pl.ds(r0, RPW)], idxv)
        pltpu.async_copy(table_r.at[idxv], rows, sem).wait()
        if with_base:
            pltpu.sync_copy(base_r.at[pl.ds(r0, RPW)], bbuf)
            _rows_binop(rows, bbuf, RPW, d, mul=False)
        pltpu.sync_copy(rows, out_r.at[pl.ds(r0, RPW)])

    fn = pl.kernel(
        body,
        out_type=jax.ShapeDtypeStruct((NP, d), jnp.float32),
        mesh=_MESH,
        scratch_types=scratch,
    )
    return fn(table, idx, base) if with_base else fn(table, idx)


@functools.partial(jax.jit, static_argnames=("with_wf",))
def sc_segsum_gather(table, src, dst, wf=None, *, with_wf=False):
    d = table.shape[1]
    EC = 80
    NCH = EPW // EC
    PAIRS = (NCH - 1) // 2
    scratch = [
        pltpu.VMEM((EPW,), jnp.int32),
        pltpu.VMEM((EC,), jnp.int32),
        pltpu.VMEM((EC,), jnp.int32),
        pltpu.VMEM((EC, d), jnp.float32),
        pltpu.VMEM((EC, d), jnp.float32),
        pltpu.VMEM_SHARED((NP, d), jnp.float32),
        pltpu.SemaphoreType.DMA,
        pltpu.SemaphoreType.DMA,
        pltpu.SemaphoreType.DMA,
        pltpu.SemaphoreType.DMA,
    ]
    if with_wf:
        scratch.insert(5, pltpu.VMEM((EC, d), jnp.float32))
        scratch.append(pltpu.SemaphoreType.DMA)

    def body(*refs):
        if with_wf:
            (table_r, src_r, dst_r, wf_r, out_r, sidxall, didxA, didxB,
             rowsA, rowsB, wfb, acc, gA, gB, sA, sB, wsem) = refs
        else:
            (table_r, src_r, dst_r, out_r, sidxall, didxA, didxB,
             rowsA, rowsB, acc, gA, gB, sA, sB) = refs
        cid = lax.axis_index("c")
        wid = _wid()
        e0 = wid * EPW
        pltpu.sync_copy(src_r.at[pl.ds(e0, EPW)], sidxall)
        _zero_acc_and_barrier(acc, rowsA, NP, d)

        def fire(j, didx, rows, sem):
            pltpu.async_copy(dst_r.at[pl.ds(e0 + j * EC, EC)], didx, sem)
            pltpu.async_copy(table_r.at[sidxall.at[pl.ds(j * EC, EC)]], rows, sem)

        def wait_g(didx, rows, sem):
            pltpu.make_async_copy(dst_r.at[pl.ds(0, EC)], didx, sem).wait()
            pltpu.make_async_copy(table_r.at[sidxall.at[pl.ds(0, EC)]], rows, sem).wait()

        def fire_wf(j):
            pltpu.async_copy(wf_r.at[pl.ds(e0 + j * EC, EC)], wfb, wsem)

        def wait_wf():
            pltpu.make_async_copy(wf_r.at[pl.ds(0, EC)], wfb, wsem).wait()

        def scat(didx, rows, sem):
            pltpu.async_copy(rows, acc.at[didx], sem, add=True)

        def wait_scat(didx, rows, sem):
            pltpu.make_async_copy(rows, acc.at[didx], sem).wait()

        fire(0, didxA, rowsA, gA)
        fire(1, didxB, rowsB, gB)
        if with_wf:
            fire_wf(0)

        def pair(jj, _):
            j0 = 2 * jj
            j2 = j0 + 2
            j3 = jnp.minimum(j0 + 3, NCH - 1)
            wait_g(didxA, rowsA, gA)
            if with_wf:
                wait_wf()
                _rows_binop(rowsA, wfb, EC, d, mul=True)
            scat(didxA, rowsA, sA)
            if with_wf:
                fire_wf(j0 + 1)
            wait_scat(didxA, rowsA, sA)
            fire(j2, didxA, rowsA, gA)
            wait_g(didxB, rowsB, gB)
            if with_wf:
                wait_wf()
                _rows_binop(rowsB, wfb, EC, d, mul=True)
            scat(didxB, rowsB, sB)
            if with_wf:
                fire_wf(j2)
            wait_scat(didxB, rowsB, sB)
            fire(j3, didxB, rowsB, gB)
            return 0

        lax.fori_loop(0, PAIRS, pair, 0)
        wait_g(didxA, rowsA, gA)
        if with_wf:
            wait_wf()
            _rows_binop(rowsA, wfb, EC, d, mul=True)
        scat(didxA, rowsA, sA)
        wait_scat(didxA, rowsA, sA)
        wait_g(didxB, rowsB, gB)
        plsc.subcore_barrier()
        _acc_readout(acc, rowsA, out_r, cid, NP, d)

    fn = pl.kernel(
        body,
        out_type=jax.ShapeDtypeStruct((NC, NP, d), jnp.float32),
        mesh=_MESH,
        scratch_types=scratch,
    )
    return fn(table, src, dst, wf) if with_wf else fn(table, src, dst)


@jax.jit
def sc_scatter_rows(values, dst):
    d = values.shape[1]

    def body(val_r, dst_r, out_r, didx, rows, acc, sem):
        cid = lax.axis_index("c")
        wid = _wid()
        _zero_acc_and_barrier(acc, rows, GP, d)
        r0 = wid * RPW
        pltpu.sync_copy(val_r.at[pl.ds(r0, RPW)], rows)
        pltpu.sync_copy(dst_r.at[pl.ds(r0, RPW)], didx)
        pltpu.sync_copy(rows, acc.at[didx], add=True)
        plsc.subcore_barrier()
        _acc_readout(acc, rows, out_r, cid, GP, d)

    fn = pl.kernel(
        body,
        out_type=jax.ShapeDtypeStruct((NC, GP, d), jnp.float32),
        mesh=_MESH,
        scratch_types=[
            pltpu.VMEM((RPW,), jnp.int32),
            pltpu.VMEM((RPW, d), jnp.float32),
            pltpu.VMEM_SHARED((GP, d), jnp.float32),
            pltpu.SemaphoreType.DMA,
        ],
    )
    return fn(values, dst)


@jax.jit
def sc_dist2(px, py, pz, src, dst):
    def body(px_r, py_r, pz_r, src_r, dst_r, out_r,
             sidx, didx, ax, ay, az, bx, by, bz, outb, sem):
        wid = _wid()
        e0 = wid * EPW
        pltpu.sync_copy(src_r.at[pl.ds(e0, EPW)], sidx)
        pltpu.sync_copy(dst_r.at[pl.ds(e0, EPW)], didx)
        plan = ((px_r, sidx, ax), (py_r, sidx, ay), (pz_r, sidx, az),
                (px_r, didx, bx), (py_r, didx, by), (pz_r, didx, bz))
        descs = [pltpu.async_copy(tab.at[idx], buf, sem) for tab, idx, buf in plan]
        for desc in descs:
            desc.wait()

        def step(i, _):
            sl = pl.ds(i * 16, 16)
            dx = ax[sl] - bx[sl]
            dy = ay[sl] - by[sl]
            dz = az[sl] - bz[sl]
            outb[sl] = dx * dx + dy * dy + dz * dz + 1e-12
            return 0

        lax.fori_loop(0, EPW // 16, step, 0)
        pltpu.sync_copy(outb, out_r.at[pl.ds(e0, EPW)])

    fn = pl.kernel(
        body,
        out_type=jax.ShapeDtypeStruct((E,), jnp.float32),
        mesh=_MESH,
        scratch_types=[
            pltpu.VMEM((EPW,), jnp.int32),
            pltpu.VMEM((EPW,), jnp.int32),
            pltpu.VMEM((EPW,), jnp.float32),
            pltpu.VMEM((EPW,), jnp.float32),
            pltpu.VMEM((EPW,), jnp.float32),
            pltpu.VMEM((EPW,), jnp.float32),
            pltpu.VMEM((EPW,), jnp.float32),
            pltpu.VMEM((EPW,), jnp.float32),
            pltpu.VMEM((EPW,), jnp.float32),
            pltpu.SemaphoreType.DMA,
        ],
    )
    return fn(px, py, pz, src, dst)


def _ssp(x):
    return jax.nn.softplus(x) - math.log(2.0)


def _dot(a, b):
    return jnp.dot(a, b, preferred_element_type=jnp.float32)


BN = 2048


@functools.partial(jax.jit, static_argnames=("relu_out",))
def tc_gin_layer(p0, p1, hin, c0, c1, comb, w1, b1, w2, b2, *, relu_out):
    def body(p0r, p1r, hr, c0r, c1r, cbr, w1r, b1r, w2r, b2r, outr):
        agg = p0r[...] + p1r[...] + hr[...] + _dot(c0r[...] + c1r[...], cbr[...])
        hmid = jax.nn.relu(_dot(agg, w1r[...]) + b1r[...])
        res = _dot(hmid, w2r[...]) + b2r[...]
        outr[...] = jax.nn.relu(res) if relu_out else res

    nb = NP // BN
    full = lambda i: (0, 0)
    row = lambda i: (i, 0)
    return pl.pallas_call(
        body,
        grid=(nb,),
        in_specs=[
            pl.BlockSpec((BN, D), row),
            pl.BlockSpec((BN, D), row),
            pl.BlockSpec((BN, D), row),
            pl.BlockSpec((BN, D), row),
            pl.BlockSpec((BN, D), row),
            pl.BlockSpec((D, D), full),
            pl.BlockSpec((D, 2 * D), full),
            pl.BlockSpec((1, 2 * D), full),
            pl.BlockSpec((2 * D, D), full),
            pl.BlockSpec((1, D), full),
        ],
        out_specs=pl.BlockSpec((BN, D), row),
        out_shape=jax.ShapeDtypeStruct((NP, D), jnp.float32),
    )(p0, p1, hin, c0, c1, comb, w1, b1, w2, b2)


@jax.jit
def tc_vn_mlp(s0, s1, vn, w1, b1, w2, b2):
    def body(s0r, s1r, vnr, w1r, b1r, w2r, b2r, outr):
        vin = s0r[...] + s1r[...] + vnr[...]
        vmid = jax.nn.relu(_dot(vin, w1r[...]) + b1r[...])
        outr[...] = jax.nn.relu(_dot(vmid, w2r[...]) + b2r[...])

    return pl.pallas_call(
        body,
        out_shape=jax.ShapeDtypeStruct((G, D), jnp.float32),
    )(s0, s1, vn, w1, b1, w2, b2)


BE = 4000


@jax.jit
def tc_filter(dist2, offs, mw1, mb1, mw2, mb2):
    coeff = -0.5 / (CUT / (NG - 1)) ** 2

    def body(dr, offr, w1r, b1r, w2r, b2r, outr):
        dist = jnp.sqrt(dr[...])
        ea = jnp.exp(coeff * (dist - offr[...]) ** 2)
        mid = _ssp(_dot(ea, w1r[...]) + b1r[...])
        wf = _dot(mid, w2r[...]) + b2r[...]
        cw = 0.5 * (jnp.cos(dist * (math.pi / CUT)) + 1.0)
        cw = cw * (dist < CUT).astype(jnp.float32)
        outr[...] = wf * cw

    nb = E // BE
    full = lambda i: (0, 0)
    row = lambda i: (i, 0)
    return pl.pallas_call(
        body,
        grid=(nb,),
        in_specs=[
            pl.BlockSpec((BE, 1), row),
            pl.BlockSpec((1, 64), full),
            pl.BlockSpec((64, H), full),
            pl.BlockSpec((1, H), full),
            pl.BlockSpec((H, H), full),
            pl.BlockSpec((1, H), full),
        ],
        out_specs=pl.BlockSpec((BE, H), row),
        out_shape=jax.ShapeDtypeStruct((E, H), jnp.float32),
    )(dist2, offs, mw1, mb1, mw2, mb2)


@jax.jit
def tc_matmul(x, w):
    def body(xr, wr, outr):
        outr[...] = _dot(xr[...], wr[...])

    nb = NP // BN
    return pl.pallas_call(
        body,
        grid=(nb,),
        in_specs=[
            pl.BlockSpec((BN, D), lambda i: (i, 0)),
            pl.BlockSpec((D, D), lambda i: (0, 0)),
        ],
        out_specs=pl.BlockSpec((BN, D), lambda i: (i, 0)),
        out_shape=jax.ShapeDtypeStruct((NP, D), jnp.float32),
    )(x, w)


@jax.jit
def tc_sch_update(a0, a1, hs, w1, b1, w2, b2):
    def body(a0r, a1r, hsr, w1r, b1r, w2r, b2r, outr):
        x2 = _ssp(_dot(a0r[...] + a1r[...], w1r[...]) + b1r[...])
        outr[...] = hsr[...] + _dot(x2, w2r[...]) + b2r[...]

    nb = NP // BN
    full = lambda i: (0, 0)
    row = lambda i: (i, 0)
    return pl.pallas_call(
        body,
        grid=(nb,),
        in_specs=[
            pl.BlockSpec((BN, H), row),
            pl.BlockSpec((BN, H), row),
            pl.BlockSpec((BN, H), row),
            pl.BlockSpec((H, H), full),
            pl.BlockSpec((1, H), full),
            pl.BlockSpec((H, H), full),
            pl.BlockSpec((1, H), full),
        ],
        out_specs=pl.BlockSpec((BN, H), row),
        out_shape=jax.ShapeDtypeStruct((NP, H), jnp.float32),
    )(a0, a1, hs, w1, b1, w2, b2)


@jax.jit
def tc_head(h2a, h2b, h3a, h3b, cb0, cb1,
            p2w, p2b, p3w, p3b, gw1, gb1, gw2, gb2, cw1, clb1, cw2, clb2):
    def body(h2ar, h2br, h3ar, h3br, c0r, c1r,
             p2wr, p2br, p3wr, p3br, gw1r, gb1r, gw2r, gb2r,
             cw1r, clb1r, cw2r, clb2r, outr):
        cnt = jnp.maximum((c0r[...] + c1r[...])[:, 0:1], 1.0)
        h2d = (h2ar[...] + h2br[...]) / cnt
        h3d = (h3ar[...] + h3br[...]) / cnt
        h2p = _dot(h2d, p2wr[...]) + p2br[...]
        h3p = _dot(h3d, p3wr[...]) + p3br[...]
        cat = jnp.concatenate([h2p, h3p], axis=1)
        gmid = jax.nn.relu(_dot(cat, gw1r[...]) + gb1r[...])
        a = jax.nn.sigmoid(_dot(gmid, gw2r[...]) + gb2r[...])
        hf = a * h2p + (1.0 - a) * h3p
        cm = jax.nn.relu(_dot(hf, cw1r[...]) + clb1r[...])
        outr[...] = _dot(cm, cw2r[...]) + clb2r[...]

    return pl.pallas_call(
        body,
        out_shape=jax.ShapeDtypeStruct((G, NT), jnp.float32),
    )(h2a, h2b, h3a, h3b, cb0, cb1,
      p2w, p2b, p3w, p3b, gw1, gb1, gw2, gb2, cw1, clb1, cw2, clb2)


def kernel(x_2d, edge_index_2d, edge_attr_2d, batch_2d, z_3d, pos_3d,
           batch_3d, edge_index_3d, params):
    p = params
    npad = NP - N

    atomcomb = (p['atom_emb1'][:, None, :] + p['atom_emb2'][None, :, :])
    atomcomb = jnp.pad(atomcomb.reshape(360, D), ((0, 8), (0, 0)))
    aidx = jnp.pad(x_2d[:, 0] * 3 + x_2d[:, 1], (0, npad), constant_values=360)
    aidx = aidx.astype(jnp.int32)
    s2 = edge_index_2d[0].astype(jnp.int32)
    d2 = edge_index_2d[1].astype(jnp.int32)
    s3 = edge_index_3d[0].astype(jnp.int32)
    d3 = edge_index_3d[1].astype(jnp.int32)
    t18 = (edge_attr_2d[:, 0] * 3 + edge_attr_2d[:, 1]).astype(jnp.int32)
    eye128 = jnp.eye(128, dtype=jnp.float32)
    b2g = jnp.pad(batch_2d, (0, npad)).astype(jnp.int32)
    b2s = jnp.pad(batch_2d, (0, npad), constant_values=G).astype(jnp.int32)
    ztab = jnp.pad(p['z_emb'], ((0, 12), (0, 0)))
    zidx = jnp.pad(z_3d, (0, npad), constant_values=100).astype(jnp.int32)
    pos_pad = jnp.pad(pos_3d, ((0, npad), (0, 0)))
    px = jnp.asarray(pos_pad[:, 0])
    py = jnp.asarray(pos_pad[:, 1])
    pz = jnp.asarray(pos_pad[:, 2])
    ones128 = jnp.ones((NP, D), jnp.float32)
    offs = jnp.linspace(0.0, CUT, NG).astype(jnp.float32)
    offs64 = jnp.pad(offs, (0, 64 - NG), constant_values=CUT).reshape(1, 64)

    def b2d(v):
        return v.reshape(1, -1)

    h = sc_gather(atomcomb, aidx)
    cnt = sc_segsum_gather(eye128, t18, d2)
    vn = jnp.zeros((G, D), jnp.float32)
    for l in range(L):
        comb = (p['gin%d_bond1' % l][:, None, :] + p['gin%d_bond2' % l][None, :, :])
        comb128 = jnp.pad(comb.reshape(18, D), ((0, 110), (0, 0)))
        if l == 0:
            hin = h
        else:
            hin = sc_gather(vn, b2g, h, with_base=True)
        seg = sc_segsum_gather(hin, s2, d2)
        h = tc_gin_layer(seg[0], seg[1], hin, cnt[0], cnt[1], comb128,
                         p['gin%d_W1' % l], b2d(p['gin%d_b1' % l]),
                         p['gin%d_W2' % l], b2d(p['gin%d_b2' % l]),
                         relu_out=(l < L - 1))
        if l < L - 1:
            vseg = sc_scatter_rows(h, b2s)
            vn = tc_vn_mlp(vseg[0, :G], vseg[1, :G], vn,
                           p['vn%d_W1' % l], b2d(p['vn%d_b1' % l]),
                           p['vn%d_W2' % l], b2d(p['vn%d_b2' % l]))
    poolh = sc_scatter_rows(h, b2s)
    cntb = sc_scatter_rows(ones128, b2s)

    hs = sc_gather(ztab, zidx)
    dist2 = sc_dist2(px, py, pz, s3, d3).reshape(E, 1)
    for i in range(NI):
        mw1 = jnp.pad(p['sch%d_mW1' % i], ((0, 64 - NG), (0, 0)))
        wf = tc_filter(dist2, offs64, mw1, b2d(p['sch%d_mb1' % i]),
                       p['sch%d_mW2' % i], b2d(p['sch%d_mb2' % i]))
        x1 = tc_matmul(hs, p['sch%d_lin1' % i])
        seg = sc_segsum_gather(x1, s3, d3, wf, with_wf=True)
        hs = tc_sch_update(seg[0], seg[1], hs,
                           p['sch%d_lin2W' % i], b2d(p['sch%d_lin2b' % i]),
                           p['sch%d_linW' % i], b2d(p['sch%d_linb' % i]))
    poolhs = sc_scatter_rows(hs, b2s)

    out = tc_head(poolh[0, :G], poolh[1, :G], poolhs[0, :G], poolhs[1, :G],
                  cntb[0, :G], cntb[1, :G],
                  p['proj2d_W'], b2d(p['proj2d_b']),
                  p['proj3d_W'], b2d(p['proj3d_b']),
                  p['gate_W1'], b2d(p['gate_b1']),
                  p['gate_W2'], b2d(p['gate_b2']),
                  p['cls_W1'], b2d(p['cls_b1']),
                  p['cls_W2'], b2d(p['cls_b2']))
    return out
